# baseline (speedup 1.0000x reference)
import jax
import jax.numpy as jnp
from jax import lax
from jax.experimental import pallas as pl
from jax.experimental.pallas import tpu as pltpu
from jax.experimental.pallas import tpu_sc as plsc

_B, _C, _L, _N = 16, 512, 4096, 1024
_GAMMA = 1.0
_CW = 16
_HC = 4
_NQ = _CW // _HC
_LANES = 16
_NG = _N // _LANES
_NSUB = 16
_CCORE = _NSUB * _CW


def _sampler_body(inp, loc_in, out, loc_all, i0_v, w1_v, inb0, inb1,
                  outb, tmp, mrg, shm, sem0, sem1):
    cid = lax.axis_index("c")
    sid = lax.axis_index("s")
    wid = cid * _NSUB + sid
    c0 = wid * _CW

    def run_idx_loop(b):
        @plsc.parallel_loop(0, _NG, unroll=2)
        def idx_body(j):
            loc = loc_all[pl.ds(b * _N + j * _LANES, _LANES)]
            i0 = loc.astype(jnp.int32)
            i0_v[pl.ds(j * _LANES, _LANES)] = i0
            w1_v[pl.ds(j * _LANES, _LANES)] = loc - i0.astype(jnp.float32)

    def compute_quarter(buf, q):
        @plsc.parallel_loop(0, _NG, unroll=8)
        def grp_body(g):
            n_base = g * _LANES
            sl = pl.ds(n_base, _LANES)
            i0 = i0_v[sl]
            w1 = w1_v[sl]
            i1 = jnp.minimum(i0 + 1, _L - 1)
            w0 = 1.0 - w1
            for c in range(_HC):
                c_idx = jnp.full((_LANES,), c, jnp.int32)
                v0 = plsc.load_gather(buf, [c_idx, i0])
                v1 = plsc.load_gather(buf, [c_idx, i1])
                outb[q * _HC + c, sl] = w0 * v0 + w1 * v1

    def in_slab(b, q):
        return inp.at[b, pl.ds(c0 + q * _HC, _HC)]

    bufs = (inb0, inb1)
    sems = (sem0, sem1)

    pltpu.sync_copy(loc_in, loc_all)
    pltpu.async_copy(in_slab(0, 0), inb0, sem0)

    r_slab = sid // 2
    wh = sid % 2

    def per_batch(b, _):
        with jax.named_scope("idx_phase"):
            run_idx_loop(b)

        for q in range(_NQ):
            if q + 1 < _NQ:
                pltpu.async_copy(in_slab(b, q + 1),
                                 bufs[(q + 1) % 2], sems[(q + 1) % 2])
            else:
                @pl.when(b + 1 < _B)
                def _():
                    pltpu.async_copy(in_slab(b + 1, 0), bufs[0], sems[0])

            with jax.named_scope("in_wait"):
                pltpu.make_async_copy(in_slab(b, q), bufs[q % 2],
                                      sems[q % 2]).wait()
            with jax.named_scope("gather"):
                compute_quarter(bufs[q % 2], q)

        with jax.named_scope("mrg_put"):
            pltpu.sync_copy(outb, shm.at[sid])
        with jax.named_scope("mrg_bar1"):
            plsc.subcore_barrier()
        with jax.named_scope("mrg_get"):
            pltpu.sync_copy(
                shm.at[pl.ds(wh * 8, 8), :, pl.ds(r_slab * 128, 128)], tmp)
        with jax.named_scope("mrg_ilv"):
            @plsc.parallel_loop(0, 128, unroll=2)
            def row_body(n):
                n_idx = jnp.full((_LANES,), n, jnp.int32)
                ci = lax.iota(jnp.int32, _LANES)
                for j in range(8):
                    v = plsc.load_gather(
                        tmp, [jnp.full((_LANES,), j, jnp.int32), ci, n_idx])
                    mrg[n, pl.ds(j * _CW, _LANES)] = v
        with jax.named_scope("out_dma"):
            pltpu.sync_copy(
                mrg,
                out.at[b, pl.ds(r_slab * 128, 128),
                       pl.ds(cid * _CCORE + wh * 128, 128)],
            )
        with jax.named_scope("mrg_bar2"):
            plsc.subcore_barrier()
        return 0

    lax.fori_loop(0, _B, per_batch, 0)


def kernel(input, point, offset):
    loc = jnp.clip(point[:, :, 0] + _GAMMA * offset[:, :, 0], 0.0,
                   float(_L - 1)).reshape(_B * _N)
    mesh = plsc.VectorSubcoreMesh(core_axis_name="c", subcore_axis_name="s")
    f = pl.kernel(
        _sampler_body,
        out_type=jax.ShapeDtypeStruct((_B, _N, _C), jnp.float32),
        mesh=mesh,
        scratch_types=[
            pltpu.VMEM((_B * _N,), jnp.float32),
            pltpu.VMEM((_N,), jnp.int32),
            pltpu.VMEM((_N,), jnp.float32),
            pltpu.VMEM((_HC, _L), jnp.float32),
            pltpu.VMEM((_HC, _L), jnp.float32),
            pltpu.VMEM((_CW, _N), jnp.float32),
            pltpu.VMEM((8, _CW, 128), jnp.float32),
            pltpu.VMEM((128, 128), jnp.float32),
            pltpu.VMEM_SHARED((_NSUB, _CW, _N), jnp.float32),
            pltpu.SemaphoreType.DMA,
            pltpu.SemaphoreType.DMA,
        ],
        compiler_params=pltpu.CompilerParams(needs_layout_passes=False),
    )
    return f(input, loc)
